# baseline (speedup 1.0000x reference)
import functools

import jax
import jax.numpy as jnp
from jax.experimental import pallas as pl
from jax.experimental.pallas import tpu as pltpu

_B = 8
_CIN = 128
_S = 4
_LANE = 128
_THW = 512


def _fused_kernel(w1_ref, b1_ref, x0_ref, x1_ref, x2_ref, x3_ref,
                  o0_ref, o1_ref, o2_ref, o3_ref,
                  acc_ref, wts_ref, *, ht, thw, hw, wdim):
    i = pl.program_id(0)
    x_refs = (x0_ref, x1_ref, x2_ref, x3_ref)
    o_refs = (o0_ref, o1_ref, o2_ref, o3_ref)

    @pl.when(i == 0)
    def _():
        acc_ref[...] = jnp.zeros_like(acc_ref)

    @pl.when(i < ht)
    def _():
        for s in range(_S):
            x = x_refs[s][...].astype(jnp.float32)
            part = x[:, 0:_LANE]
            for j in range(1, thw // _LANE):
                part = part + x[:, j * _LANE:(j + 1) * _LANE]
            acc_ref[s] = acc_ref[s] + part

    @pl.when(i == ht)
    def _():
        b1 = b1_ref[...]
        wts = jnp.broadcast_to(b1, (_B, _CIN)).astype(jnp.float32)
        for s in range(_S):
            m_s = jnp.sum(acc_ref[s].reshape(_B, _CIN, _LANE), axis=2)
            m_s = m_s * (1.0 / hw)
            w1_s = w1_ref[:, s * _CIN:(s + 1) * _CIN]
            wts = wts + jax.lax.dot_general(
                m_s, w1_s, (((1,), (1,)), ((), ())),
                preferred_element_type=jnp.float32)
        wts_ref[...] = wts

    @pl.when(i >= ht)
    def _():
        w = wts_ref[...].astype(jnp.bfloat16)
        th = thw // wdim
        for s in range(_S):
            for k in range(_B):
                xk = x_refs[s][k * _CIN:(k + 1) * _CIN, :]
                res = jnp.dot(w, xk, preferred_element_type=jnp.float32)
                o_refs[s][k] = res.reshape(_B, th, wdim)


def kernel(x0, x1, x2, x3, w1, b1):
    B, cin, H, W = x0.shape
    HW = H * W
    R = B * cin
    thw = min(_THW, HW)
    ht = HW // thw
    xs_c = [x.reshape(R, HW).astype(jnp.bfloat16) for x in (x0, x1, x2, x3)]

    body = functools.partial(_fused_kernel, ht=ht, thw=thw, hw=HW, wdim=W)

    def in_idx(i):
        return (0, jnp.where(i < ht, i, i - ht))

    def out_idx(i):
        return (0, 0, jnp.where(i < ht, 0, i - ht), 0)

    th = thw // W
    out_shape = jax.ShapeDtypeStruct((B, B, H, W), jnp.float32)
    outs = pl.pallas_call(
        body,
        out_shape=[out_shape] * _S,
        grid=(2 * ht,),
        in_specs=[
            pl.BlockSpec((cin, _S * cin), lambda i: (0, 0)),
            pl.BlockSpec((1, cin), lambda i: (0, 0)),
        ] + [pl.BlockSpec((R, thw), in_idx)] * _S,
        out_specs=[pl.BlockSpec((B, B, th, W), out_idx)] * _S,
        scratch_shapes=[
            pltpu.VMEM((_S, R, _LANE), jnp.float32),
            pltpu.VMEM((_B, _CIN), jnp.float32),
        ],
        compiler_params=pltpu.CompilerParams(
            dimension_semantics=("arbitrary",),
            vmem_limit_bytes=48 * 1024 * 1024,
        ),
    )(w1, b1.reshape(1, cin), *xs_c)

    return list(outs)


# speedup vs baseline: 4.2820x; 4.2820x over previous
import functools

import jax
import jax.numpy as jnp
from jax.experimental import pallas as pl
from jax.experimental.pallas import tpu as pltpu

_B = 8
_CIN = 128
_S = 4
_THW = 512


def _fused_kernel(w1_ref, b1_ref, x0_ref, x1_ref, x2_ref, x3_ref,
                  o0_ref, o1_ref, o2_ref, o3_ref,
                  acc_ref, wts_ref, *, ht, th, wdim, hw):
    i = pl.program_id(0)
    x_refs = (x0_ref, x1_ref, x2_ref, x3_ref)
    o_refs = (o0_ref, o1_ref, o2_ref, o3_ref)
    groups = th * wdim // _B

    @pl.when(i == 0)
    def _():
        acc_ref[...] = jnp.zeros_like(acc_ref)

    @pl.when(i < ht)
    def _():
        for s in range(_S):
            x4 = x_refs[s][...]
            part = x4.reshape(_B, groups, _B, _CIN).sum(axis=1)
            acc_ref[s] = acc_ref[s] + part

    @pl.when(i == ht)
    def _():
        b1 = b1_ref[...]
        wts = jnp.broadcast_to(b1, (_B, _CIN)).astype(jnp.float32)
        for s in range(_S):
            m_s = jnp.sum(acc_ref[s], axis=1) * (1.0 / hw)
            w1_s = w1_ref[:, s * _CIN:(s + 1) * _CIN]
            wts = wts + jax.lax.dot_general(
                m_s, w1_s, (((1,), (1,)), ((), ())),
                preferred_element_type=jnp.float32)
        wts_ref[...] = wts

    @pl.when(i >= ht)
    def _():
        w = wts_ref[...]
        for s in range(_S):
            for k in range(_B):
                xk = x_refs[s][k].reshape(th * wdim, _CIN)
                res = jax.lax.dot_general(
                    w, xk, (((1,), (1,)), ((), ())),
                    preferred_element_type=jnp.float32)
                o_refs[s][k] = res.reshape(_B, th, wdim)


def kernel(x0, x1, x2, x3, w1, b1):
    B, cin, H, W = x0.shape
    HW = H * W
    thw = min(_THW, HW)
    ht = HW // thw
    th = thw // W
    xs_t = [x.transpose(0, 2, 3, 1) for x in (x0, x1, x2, x3)]

    body = functools.partial(_fused_kernel, ht=ht, th=th, wdim=W, hw=HW)

    def in_idx(i):
        return (0, jnp.where(i < ht, i, i - ht), 0, 0)

    def out_idx(i):
        return (0, 0, jnp.where(i < ht, 0, i - ht), 0)

    out_shape = jax.ShapeDtypeStruct((B, B, H, W), jnp.float32)
    outs = pl.pallas_call(
        body,
        out_shape=[out_shape] * _S,
        grid=(2 * ht,),
        in_specs=[
            pl.BlockSpec((cin, _S * cin), lambda i: (0, 0)),
            pl.BlockSpec((1, cin), lambda i: (0, 0)),
        ] + [pl.BlockSpec((B, th, W, cin), in_idx)] * _S,
        out_specs=[pl.BlockSpec((B, B, th, W), out_idx)] * _S,
        scratch_shapes=[
            pltpu.VMEM((_S, B, _B, cin), jnp.float32),
            pltpu.VMEM((_B, cin), jnp.float32),
        ],
        compiler_params=pltpu.CompilerParams(
            dimension_semantics=("arbitrary",),
            vmem_limit_bytes=56 * 1024 * 1024,
        ),
    )(w1, b1.reshape(1, cin), *xs_t)

    return list(outs)


# speedup vs baseline: 5.7121x; 1.3340x over previous
import functools

import jax
import jax.numpy as jnp
from jax.experimental import pallas as pl
from jax.experimental.pallas import tpu as pltpu

_B = 8
_CIN = 128
_S = 4
_THW = 512


def _fused_kernel(w1_ref, b1_ref, x0_ref, x1_ref, x2_ref, x3_ref,
                  o0_ref, o1_ref, o2_ref, o3_ref,
                  acc_ref, wts_ref, xbuf_ref, *, ht, th, wdim, hw):
    i = pl.program_id(0)
    x_refs = (x0_ref, x1_ref, x2_ref, x3_ref)
    o_refs = (o0_ref, o1_ref, o2_ref, o3_ref)
    groups = th * wdim // _B

    @pl.when(i == 0)
    def _():
        acc_ref[...] = jnp.zeros_like(acc_ref)

    @pl.when(i < ht)
    def _():
        for s in range(_S):
            x4 = x_refs[s][...]
            part = x4.reshape(_B, groups, _B, _CIN).sum(axis=1)
            acc_ref[s] = acc_ref[s] + part
            xbuf_ref[s, :, pl.ds(i * th, th)] = x4.astype(jnp.bfloat16)

    @pl.when(i == ht)
    def _():
        b1 = b1_ref[...]
        wts = jnp.broadcast_to(b1, (_B, _CIN)).astype(jnp.float32)
        for s in range(_S):
            m_s = jnp.sum(acc_ref[s], axis=1) * (1.0 / hw)
            w1_s = w1_ref[:, s * _CIN:(s + 1) * _CIN]
            wts = wts + jax.lax.dot_general(
                m_s, w1_s, (((1,), (1,)), ((), ())),
                preferred_element_type=jnp.float32)
        wts_ref[...] = wts

    @pl.when(i >= ht)
    def _():
        w = wts_ref[...].astype(jnp.bfloat16)
        j = i - ht
        for s in range(_S):
            for k in range(_B):
                xk = xbuf_ref[s, k, pl.ds(j * th, th)].reshape(
                    th * wdim, _CIN)
                res = jax.lax.dot_general(
                    w, xk, (((1,), (1,)), ((), ())),
                    preferred_element_type=jnp.float32)
                o_refs[s][k] = res.reshape(_B, th, wdim)


def kernel(x0, x1, x2, x3, w1, b1):
    B, cin, H, W = x0.shape
    HW = H * W
    thw = min(_THW, HW)
    ht = HW // thw
    th = thw // W
    xs_t = [x.transpose(0, 2, 3, 1) for x in (x0, x1, x2, x3)]

    body = functools.partial(_fused_kernel, ht=ht, th=th, wdim=W, hw=HW)

    def in_idx(i):
        return (0, jnp.where(i < ht, i, ht - 1), 0, 0)

    def out_idx(i):
        return (0, 0, jnp.where(i < ht, 0, i - ht), 0)

    out_shape = jax.ShapeDtypeStruct((B, B, H, W), jnp.float32)
    outs = pl.pallas_call(
        body,
        out_shape=[out_shape] * _S,
        grid=(2 * ht,),
        in_specs=[
            pl.BlockSpec((cin, _S * cin), lambda i: (0, 0)),
            pl.BlockSpec((1, cin), lambda i: (0, 0)),
        ] + [pl.BlockSpec((B, th, W, cin), in_idx)] * _S,
        out_specs=[pl.BlockSpec((B, B, th, W), out_idx)] * _S,
        scratch_shapes=[
            pltpu.VMEM((_S, B, _B, cin), jnp.float32),
            pltpu.VMEM((_B, cin), jnp.float32),
            pltpu.VMEM((_S, B, H, W, cin), jnp.bfloat16),
        ],
        compiler_params=pltpu.CompilerParams(
            dimension_semantics=("arbitrary",),
            vmem_limit_bytes=56 * 1024 * 1024,
        ),
    )(w1, b1.reshape(1, cin), *xs_t)

    return list(outs)
